# baseline (speedup 1.0000x reference)
import jax
import jax.numpy as jnp
from jax import lax
from jax.experimental import pallas as pl
from jax.experimental.pallas import tpu as pltpu
from jax.experimental.pallas import tpu_sc as plsc

_B, _N, _H, _TBL = 16, 512, 8, 300
_NC, _NS, _L = 2, 16, 16
_NW = _NC * _NS
_BLK = 128
_NBLK = _N // _BLK
_JOBS = _B * _NBLK * _NBLK // _NW
_TPAD = 520
_QR = _BLK // 4
_TCR = 8


def _decode(g):
    b = g // (_NBLK * _NBLK)
    blk = g % (_NBLK * _NBLK)
    i0 = pl.multiple_of((blk // _NBLK) * _BLK, _BLK)
    j0 = pl.multiple_of((blk % _NBLK) * _BLK, _BLK)
    return b, i0, j0


def _sc_body(idx_hbm, gcol_hbm, out_g_hbm, gcol_v, idx_v, obuf,
             sem_in, sem_out):
    wid = lax.axis_index("s") * _NC + lax.axis_index("c")
    pltpu.sync_copy(gcol_hbm, gcol_v)
    lanes = lax.iota(jnp.int32, _L)
    zeros = jnp.zeros((_L,), jnp.int32)

    def block_src(g):
        b, i0, j0 = _decode(g)
        return idx_hbm.at[b, pl.ds(i0, _BLK), pl.ds(j0, _BLK)]

    for par in range(2):
        pltpu.async_copy(block_src(par * _NW + wid), idx_v[par], sem_in[par])

    def gather_chunk(idxb, buf, q):
        @plsc.parallel_loop(0, _QR * _BLK // _L, step=1, unroll=8)
        def _w(w):
            a = w >> 3
            c = (w & 7) * _L
            sl = pl.ds(c, _L)
            row = q * _QR + a
            iv = plsc.load_gather(idxb, [c + lanes, zeros + row])
            for h in range(_H):
                buf[h, a, sl] = plsc.load_gather(gcol_v, [iv + (h * _TPAD)])

    @pl.loop(0, _JOBS, step=2)
    def _kk(kk):
        for par in range(2):
            k = kk + par
            g = k * _NW + wid
            b, i0, j0 = _decode(g)
            idxb = idx_v[par]
            pltpu.make_async_copy(block_src(g), idxb, sem_in[par]).wait()

            for q in range(4):
                p = q % 2
                buf = obuf[p]
                dst = out_g_hbm.at[
                    b, :,
                    pl.ds(pl.multiple_of(j0 + q * _QR, _QR), _QR),
                    pl.ds(i0, _BLK)]

                def _drain(buf=buf, dst=dst, p=p):
                    pltpu.make_async_copy(buf, dst, sem_out[p]).wait()
                if par == 0 and q < 2:
                    pl.when(k > 0)(_drain)
                else:
                    _drain()
                gather_chunk(idxb, buf, q)
                pltpu.async_copy(buf, dst, sem_out[p])

            @pl.when(k + 2 < _JOBS)
            def _pf():
                pltpu.async_copy(
                    block_src((k + 2) * _NW + wid), idxb, sem_in[par])

    for p in range(2):
        pltpu.make_async_copy(
            obuf[p],
            out_g_hbm.at[_B - 1, :, pl.ds(0, _QR), pl.ds(0, _BLK)],
            sem_out[p]).wait()


def _tc_body(idx_ref, tbl_ref, out_ref):
    for c in range(_N // 128):
        idxc = idx_ref[0, :, pl.ds(c * 128, 128)]
        low = jnp.bitwise_and(idxc, 127)
        hi = jnp.right_shift(idxc, 7)
        m1 = hi == 1
        m2 = hi == 2
        for h in range(_H):
            def chunk(kc):
                t = jnp.broadcast_to(
                    tbl_ref[h:h + 1, pl.ds(kc * 128, 128)], (_TCR, 128))
                return jnp.take_along_axis(
                    t, low, axis=1, mode="promise_in_bounds")
            r = jnp.where(m2, chunk(2), jnp.where(m1, chunk(1), chunk(0)))
            out_ref[0, h, :, pl.ds(c * 128, 128)] = r


@jax.jit
def kernel(spatial_pos, smiles_table, graph_table):
    srow = jnp.zeros((_H, _N), jnp.float32).at[:, :_TBL].set(smiles_table.T)
    gcol = jnp.zeros((_H, _TPAD), jnp.float32).at[:, :_TBL].set(graph_table.T)

    smiles = pl.pallas_call(
        _tc_body,
        grid=(_B, _N // _TCR),
        in_specs=[
            pl.BlockSpec((1, _TCR, _N), lambda b, s: (b, s, 0)),
            pl.BlockSpec((_H, _N), lambda b, s: (0, 0)),
        ],
        out_specs=pl.BlockSpec(
            (1, _H, _TCR, _N), lambda b, s: (b, 0, s, 0)),
        out_shape=jax.ShapeDtypeStruct((_B, _H, _N, _N), jnp.float32),
    )(spatial_pos, srow)

    mesh = plsc.VectorSubcoreMesh(core_axis_name="c", subcore_axis_name="s")
    f = pl.kernel(
        _sc_body,
        out_type=jax.ShapeDtypeStruct((_B, _H, _N, _N), jnp.float32),
        mesh=mesh,
        compiler_params=pltpu.CompilerParams(
            use_tc_tiling_on_sc=True, needs_layout_passes=False),
        scratch_types=[
            pltpu.VMEM((_H * _TPAD,), jnp.float32),
            [pltpu.VMEM((_BLK, _BLK), jnp.int32)] * 2,
            [pltpu.VMEM((_H, _QR, _BLK), jnp.float32)] * 2,
            [pltpu.SemaphoreType.DMA] * 2,
            [pltpu.SemaphoreType.DMA] * 2,
        ],
    )
    graph = f(spatial_pos, gcol.reshape(-1))
    return smiles, graph


# speedup vs baseline: 2.2247x; 2.2247x over previous
import jax
import jax.numpy as jnp
from jax import lax
from jax.experimental import pallas as pl
from jax.experimental.pallas import tpu as pltpu
from jax.experimental.pallas import tpu_sc as plsc

_B, _N, _H, _TBL = 16, 512, 8, 300
_NC, _NS, _L = 2, 16, 16
_NW = _NC * _NS
_BLK = 128
_NBLK = _N // _BLK
_JOBS = _B * _NBLK * _NBLK // _NW
_TPAD = 520
_QR = _BLK // 4


def _decode(g):
    b = g // (_NBLK * _NBLK)
    blk = g % (_NBLK * _NBLK)
    i0 = pl.multiple_of((blk // _NBLK) * _BLK, _BLK)
    j0 = pl.multiple_of((blk % _NBLK) * _BLK, _BLK)
    return b, i0, j0


def _sc_body(idx_hbm, scol_hbm, gcol_hbm, out_s_hbm, out_g_hbm,
             scol_v, gcol_v, idx_v, obuf, sem_in, sem_out):
    wid = lax.axis_index("s") * _NC + lax.axis_index("c")
    pltpu.sync_copy(scol_hbm, scol_v)
    pltpu.sync_copy(gcol_hbm, gcol_v)
    lanes = lax.iota(jnp.int32, _L)
    zeros = jnp.zeros((_L,), jnp.int32)

    def block_src(g):
        b, i0, j0 = _decode(g)
        return idx_hbm.at[b, pl.ds(i0, _BLK), pl.ds(j0, _BLK)]

    for par in range(2):
        pltpu.async_copy(block_src(par * _NW + wid), idx_v[par], sem_in[par])

    def gather_chunk(idxb, col_v, buf, q, transposed):
        @plsc.parallel_loop(0, _QR * _BLK // _L, step=1, unroll=12)
        def _w(w):
            a = w >> 3
            c = (w & 7) * _L
            sl = pl.ds(c, _L)
            row = q * _QR + a
            if transposed:
                iv = plsc.load_gather(idxb, [c + lanes, zeros + row])
            else:
                iv = idxb[row, sl]
            for h in range(_H):
                buf[h, a, sl] = plsc.load_gather(col_v, [iv + (h * _TPAD)])

    @pl.loop(0, _JOBS, step=2)
    def _kk(kk):
        for par in range(2):
            k = kk + par
            g = k * _NW + wid
            b, i0, j0 = _decode(g)
            idxb = idx_v[par]
            pltpu.make_async_copy(block_src(g), idxb, sem_in[par]).wait()

            ci = 0
            for tbl in range(2):
                col_v = (scol_v, gcol_v)[tbl]
                out_hbm = (out_s_hbm, out_g_hbm)[tbl]
                r0, c0 = ((i0, j0), (j0, i0))[tbl]
                for q in range(4):
                    p = ci % 2
                    buf = obuf[p]
                    dst = out_hbm.at[
                        b, :,
                        pl.ds(pl.multiple_of(r0 + q * _QR, _QR), _QR),
                        pl.ds(c0, _BLK)]

                    def _drain(buf=buf, dst=dst, p=p):
                        pltpu.make_async_copy(buf, dst, sem_out[p]).wait()
                    if par == 0 and ci < 2:
                        pl.when(k > 0)(_drain)
                    else:
                        _drain()
                    gather_chunk(idxb, col_v, buf, q, tbl == 1)
                    pltpu.async_copy(buf, dst, sem_out[p])
                    ci += 1

            @pl.when(k + 2 < _JOBS)
            def _pf():
                pltpu.async_copy(
                    block_src((k + 2) * _NW + wid), idxb, sem_in[par])

    for p in range(2):
        pltpu.make_async_copy(
            obuf[p],
            out_g_hbm.at[_B - 1, :, pl.ds(0, _QR), pl.ds(0, _BLK)],
            sem_out[p]).wait()


@jax.jit
def kernel(spatial_pos, smiles_table, graph_table):
    scol = jnp.zeros((_H, _TPAD), jnp.float32).at[:, :_TBL].set(smiles_table.T)
    gcol = jnp.zeros((_H, _TPAD), jnp.float32).at[:, :_TBL].set(graph_table.T)
    mesh = plsc.VectorSubcoreMesh(core_axis_name="c", subcore_axis_name="s")
    f = pl.kernel(
        _sc_body,
        out_type=(
            jax.ShapeDtypeStruct((_B, _H, _N, _N), jnp.float32),
            jax.ShapeDtypeStruct((_B, _H, _N, _N), jnp.float32),
        ),
        mesh=mesh,
        compiler_params=pltpu.CompilerParams(
            use_tc_tiling_on_sc=True, needs_layout_passes=False),
        scratch_types=[
            pltpu.VMEM((_H * _TPAD,), jnp.float32),
            pltpu.VMEM((_H * _TPAD,), jnp.float32),
            [pltpu.VMEM((_BLK, _BLK), jnp.int32)] * 2,
            [pltpu.VMEM((_H, _QR, _BLK), jnp.float32)] * 2,
            [pltpu.SemaphoreType.DMA] * 2,
            [pltpu.SemaphoreType.DMA] * 2,
        ],
    )
    return f(spatial_pos, scol.reshape(-1), gcol.reshape(-1))


# speedup vs baseline: 2.8214x; 1.2682x over previous
import jax
import jax.numpy as jnp
from jax import lax
from jax.experimental import pallas as pl
from jax.experimental.pallas import tpu as pltpu
from jax.experimental.pallas import tpu_sc as plsc

_B, _N, _H, _TBL = 16, 512, 8, 300
_NC, _NS, _L = 2, 16, 16
_NW = _NC * _NS
_BLK = 128
_NBLK = _N // _BLK
_JOBS = _B * _NBLK * _NBLK // _NW
_TPAD = 520
_QR = _BLK // 4


def _decode(g):
    b = g // (_NBLK * _NBLK)
    blk = g % (_NBLK * _NBLK)
    i0 = pl.multiple_of((blk // _NBLK) * _BLK, _BLK)
    j0 = pl.multiple_of((blk % _NBLK) * _BLK, _BLK)
    return b, i0, j0


def _sc_body(idx_hbm, scol_hbm, gcol_hbm, out_s_hbm, out_g_hbm,
             scol_v, gcol_v, idx_v, obuf, sem_in, sem_out):
    wid = lax.axis_index("s") * _NC + lax.axis_index("c")
    pltpu.sync_copy(scol_hbm, scol_v)
    pltpu.sync_copy(gcol_hbm, gcol_v)
    lanes = lax.iota(jnp.int32, _L)
    zeros = jnp.zeros((_L,), jnp.int32)

    def block_src(g):
        b, i0, j0 = _decode(g)
        return idx_hbm.at[b, pl.ds(i0, _BLK), pl.ds(j0, _BLK)]

    for par in range(2):
        pltpu.async_copy(block_src(par * _NW + wid), idx_v[par], sem_in[par])

    def gather_chunk(idxb, col_v, buf, q, transposed):
        @plsc.parallel_loop(0, _QR * _BLK // _L, step=1, unroll=8)
        def _w(w):
            a = w >> 3
            c = (w & 7) * _L
            sl = pl.ds(c, _L)
            row = q * _QR + a
            if transposed:
                iv = plsc.load_gather(idxb, [c + lanes, zeros + row])
            else:
                iv = idxb[row, sl]
            for h in range(_H):
                buf[h, a, sl] = plsc.load_gather(col_v, [iv + (h * _TPAD)])

    @pl.loop(0, _JOBS, step=2)
    def _kk(kk):
        for par in range(2):
            k = kk + par
            g = k * _NW + wid
            b, i0, j0 = _decode(g)
            idxb = idx_v[par]
            pltpu.make_async_copy(block_src(g), idxb, sem_in[par]).wait()

            ci = 0
            for tbl in range(2):
                col_v = (scol_v, gcol_v)[tbl]
                out_hbm = (out_s_hbm, out_g_hbm)[tbl]
                r0, c0 = ((i0, j0), (j0, i0))[tbl]
                for q in range(4):
                    p = ci % 2
                    buf = obuf[p]
                    dst = out_hbm.at[
                        b, :,
                        pl.ds(pl.multiple_of(r0 + q * _QR, _QR), _QR),
                        pl.ds(c0, _BLK)]

                    def _drain(buf=buf, dst=dst, p=p):
                        pltpu.make_async_copy(buf, dst, sem_out[p]).wait()
                    if par == 0 and ci < 2:
                        pl.when(k > 0)(_drain)
                    else:
                        _drain()
                    gather_chunk(idxb, col_v, buf, q, tbl == 1)
                    pltpu.async_copy(buf, dst, sem_out[p])
                    ci += 1

            @pl.when(k + 2 < _JOBS)
            def _pf():
                pltpu.async_copy(
                    block_src((k + 2) * _NW + wid), idxb, sem_in[par])

    for p in range(2):
        pltpu.make_async_copy(
            obuf[p],
            out_g_hbm.at[_B - 1, :, pl.ds(0, _QR), pl.ds(0, _BLK)],
            sem_out[p]).wait()


@jax.jit
def kernel(spatial_pos, smiles_table, graph_table):
    scol = jnp.zeros((_H, _TPAD), jnp.float32).at[:, :_TBL].set(smiles_table.T)
    gcol = jnp.zeros((_H, _TPAD), jnp.float32).at[:, :_TBL].set(graph_table.T)
    mesh = plsc.VectorSubcoreMesh(core_axis_name="c", subcore_axis_name="s")
    f = pl.kernel(
        _sc_body,
        out_type=(
            jax.ShapeDtypeStruct((_B, _H, _N, _N), jnp.float32),
            jax.ShapeDtypeStruct((_B, _H, _N, _N), jnp.float32),
        ),
        mesh=mesh,
        compiler_params=pltpu.CompilerParams(
            use_tc_tiling_on_sc=True, needs_layout_passes=False),
        scratch_types=[
            pltpu.VMEM((_H * _TPAD,), jnp.float32),
            pltpu.VMEM((_H * _TPAD,), jnp.float32),
            [pltpu.VMEM((_BLK, _BLK), jnp.int32)] * 2,
            [pltpu.VMEM((_H, _QR, _BLK), jnp.float32)] * 2,
            [pltpu.SemaphoreType.DMA] * 2,
            [pltpu.SemaphoreType.DMA] * 2,
        ],
    )
    return f(spatial_pos, scol.reshape(-1), gcol.reshape(-1))
